# Initial kernel scaffold; baseline (speedup 1.0000x reference)
import jax
import jax.numpy as jnp
from jax.experimental import pallas as pl


def kernel(pos, edge_index):
    raise NotImplementedError("write your pallas kernel here")


# speedup vs baseline: 11.7679x; 11.7679x over previous
import functools

import jax
import jax.numpy as jnp
from jax import lax
from jax.experimental import pallas as pl
from jax.experimental.pallas import tpu as pltpu
from jax.experimental.pallas import tpu_sc as plsc

LJ_SIGMA = 0.01
LJ_EPSILON = 1.0
N_NODES = 50000
N_EDGES = 1600000

NC, NS, L = 2, 16, 16
NW = NC * NS
NPAD = 50176
SLICE = NPAD // NS
SB = 128
NSB = 16
CHUNK = SB * NSB
EPAD = 1638400
E_PER_W = EPAD // NW
N_CHUNKS = E_PER_W // CHUNK

_mesh = plsc.VectorSubcoreMesh(core_axis_name="c", subcore_axis_name="s")


@functools.partial(
    pl.kernel,
    out_type=jax.ShapeDtypeStruct((NC * NPAD,), jnp.float32),
    mesh=_mesh,
    compiler_params=pltpu.CompilerParams(
        needs_layout_passes=False, use_tc_tiling_on_sc=False),
    scratch_types=[
        pltpu.VMEM((NSB, SB), jnp.int32),
        pltpu.VMEM((NSB, SB), jnp.int32),
        pltpu.VMEM((CHUNK, 4), jnp.float32),
        pltpu.VMEM((CHUNK, 4), jnp.float32),
        pltpu.VMEM((CHUNK,), jnp.float32),
        pltpu.VMEM((SLICE,), jnp.float32),
        pltpu.VMEM_SHARED((NPAD,), jnp.float32),
        pltpu.SemaphoreType.DMA,
    ],
)
def _lj_sc(pos4, srci, dsti, out, si_v, di_v, sr_v, dr_v, eng_v, stage_v,
           acc_sh, sem):
    c = lax.axis_index("c")
    s = lax.axis_index("s")
    wid = c * NS + s

    zero16 = jnp.zeros((L,), jnp.float32)

    def _zero(i, carry):
        stage_v[pl.ds(i * L, L)] = zero16
        return carry

    lax.fori_loop(0, SLICE // L, _zero, 0)
    pltpu.sync_copy(stage_v, acc_sh.at[pl.ds(s * SLICE, SLICE)])
    plsc.subcore_barrier()

    iota = lax.iota(jnp.int32, L)
    col0 = jnp.zeros((L,), jnp.int32)
    col1 = jnp.full((L,), 1, jnp.int32)
    col2 = jnp.full((L,), 2, jnp.int32)
    sig2 = jnp.full((L,), LJ_SIGMA * LJ_SIGMA, jnp.float32)
    two_eps = jnp.full((L,), 2.0 * LJ_EPSILON, jnp.float32)

    base_e = wid * E_PER_W

    def _chunk(ci, carry):
        off = base_e + ci * CHUNK
        idx_cps = []
        for j in range(NSB):
            idx_cps.append(pltpu.async_copy(
                srci.at[pl.ds(off + j * SB, SB)], si_v.at[j], sem))
            idx_cps.append(pltpu.async_copy(
                dsti.at[pl.ds(off + j * SB, SB)], di_v.at[j], sem))
        for cp in idx_cps:
            cp.wait()
        cps = []
        for j in range(NSB):
            cps.append(pltpu.async_copy(
                pos4.at[si_v.at[j]], sr_v.at[pl.ds(j * SB, SB)], sem))
            cps.append(pltpu.async_copy(
                pos4.at[di_v.at[j]], dr_v.at[pl.ds(j * SB, SB)], sem))
        for cp in cps:
            cp.wait()

        def _group(g, gcarry):
            rid = g * L + iota
            xs = plsc.load_gather(sr_v, [rid, col0])
            ys = plsc.load_gather(sr_v, [rid, col1])
            zs = plsc.load_gather(sr_v, [rid, col2])
            xd = plsc.load_gather(dr_v, [rid, col0])
            yd = plsc.load_gather(dr_v, [rid, col1])
            zd = plsc.load_gather(dr_v, [rid, col2])
            dx = xd - xs
            dy = yd - ys
            dz = zd - zs
            r2 = dx * dx + dy * dy + dz * dz
            t = sig2 / r2
            t3 = t * t * t
            eng = two_eps * (t3 * t3 - t3)
            eng_v[pl.ds(g * L, L)] = eng
            return gcarry

        lax.fori_loop(0, CHUNK // L, _group, 0)

        for j in range(NSB):
            pltpu.sync_copy(eng_v.at[pl.ds(j * SB, SB)],
                            acc_sh.at[si_v.at[j]], add=True)
        return carry

    lax.fori_loop(0, N_CHUNKS, _chunk, 0)

    plsc.subcore_barrier()
    pltpu.sync_copy(acc_sh.at[pl.ds(s * SLICE, SLICE)], stage_v)
    pltpu.sync_copy(stage_v, out.at[pl.ds(c * NPAD + s * SLICE, SLICE)])


def kernel(pos, edge_index):
    pos4 = jnp.zeros((NPAD, 4), jnp.float32)
    pos4 = pos4.at[:N_NODES, :3].set(pos.astype(jnp.float32))
    pos4 = pos4.at[N_NODES + 1, 0].set(1.0)
    npad_e = EPAD - N_EDGES
    si = jnp.concatenate([
        edge_index[0].astype(jnp.int32),
        jnp.full((npad_e,), N_NODES, jnp.int32),
    ])
    di = jnp.concatenate([
        edge_index[1].astype(jnp.int32),
        jnp.full((npad_e,), N_NODES + 1, jnp.int32),
    ])
    partial = _lj_sc(pos4, si, di)
    return (partial[:N_NODES] + partial[NPAD:NPAD + N_NODES]).reshape(
        N_NODES, 1)


# speedup vs baseline: 13.9969x; 1.1894x over previous
import functools

import jax
import jax.numpy as jnp
from jax import lax
from jax.experimental import pallas as pl
from jax.experimental.pallas import tpu as pltpu
from jax.experimental.pallas import tpu_sc as plsc

LJ_SIGMA = 0.01
LJ_EPSILON = 1.0
N_NODES = 50000
N_EDGES = 1600000

NC, NS, L = 2, 16, 16
NW = NC * NS
NPAD = 50176
SLICE = NPAD // NS
SB = 3200
NSB = 1
CHUNK = SB * NSB
N_CHUNKS = 16
E_PER_W = CHUNK * N_CHUNKS
EPAD = NW * E_PER_W

_mesh = plsc.VectorSubcoreMesh(core_axis_name="c", subcore_axis_name="s")


@functools.partial(
    pl.kernel,
    out_type=jax.ShapeDtypeStruct((NC * NPAD,), jnp.float32),
    mesh=_mesh,
    compiler_params=pltpu.CompilerParams(
        needs_layout_passes=False, use_tc_tiling_on_sc=False),
    scratch_types=[
        pltpu.VMEM((NSB, SB), jnp.int32),
        pltpu.VMEM((NSB, SB), jnp.int32),
        pltpu.VMEM((NSB, SB), jnp.int32),
        pltpu.VMEM((NSB, SB), jnp.int32),
        pltpu.VMEM((CHUNK, 4), jnp.float32),
        pltpu.VMEM((CHUNK, 4), jnp.float32),
        pltpu.VMEM((CHUNK, 4), jnp.float32),
        pltpu.VMEM((CHUNK, 4), jnp.float32),
        pltpu.VMEM((CHUNK,), jnp.float32),
        pltpu.VMEM((CHUNK,), jnp.float32),
        pltpu.VMEM((SLICE,), jnp.float32),
        pltpu.VMEM_SHARED((NPAD,), jnp.float32),
        pltpu.SemaphoreType.DMA,
        pltpu.SemaphoreType.DMA,
        pltpu.SemaphoreType.DMA,
        pltpu.SemaphoreType.DMA,
        pltpu.SemaphoreType.DMA,
        pltpu.SemaphoreType.DMA,
    ],
)
def _lj_sc(pos4, srci, dsti, out, si0, si1, di0, di1, sr0, sr1, dr0, dr1,
           en0, en1, stage_v, acc_sh, smi0, smi1, smg0, smg1, sms0, sms1):
    c = lax.axis_index("c")
    s = lax.axis_index("s")
    wid = c * NS + s

    si_v = (si0, si1)
    di_v = (di0, di1)
    sr_v = (sr0, sr1)
    dr_v = (dr0, dr1)
    en_v = (en0, en1)
    smi = (smi0, smi1)
    smg = (smg0, smg1)
    sms = (sms0, sms1)

    zero16 = jnp.zeros((L,), jnp.float32)

    def _zero(i, carry):
        stage_v[pl.ds(i * L, L)] = zero16
        return carry

    lax.fori_loop(0, SLICE // L, _zero, 0)
    pltpu.sync_copy(stage_v, acc_sh.at[pl.ds(s * SLICE, SLICE)])
    plsc.subcore_barrier()

    iota = lax.iota(jnp.int32, L)
    col0 = jnp.zeros((L,), jnp.int32)
    col1 = jnp.full((L,), 1, jnp.int32)
    col2 = jnp.full((L,), 2, jnp.int32)
    sig2 = jnp.full((L,), LJ_SIGMA * LJ_SIGMA, jnp.float32)
    two_eps = jnp.full((L,), 2.0 * LJ_EPSILON, jnp.float32)

    base_e = wid * E_PER_W

    def idx_copies(ci, b):
        off = base_e + ci * CHUNK
        cps = []
        for j in range(NSB):
            cps.append(pltpu.make_async_copy(
                srci.at[pl.ds(off + j * SB, SB)], si_v[b].at[j], smi[b]))
            cps.append(pltpu.make_async_copy(
                dsti.at[pl.ds(off + j * SB, SB)], di_v[b].at[j], smi[b]))
        return cps

    def gather_copies(b):
        cps = []
        for j in range(NSB):
            cps.append(pltpu.make_async_copy(
                pos4.at[si_v[b].at[j]], sr_v[b].at[pl.ds(j * SB, SB)],
                smg[b]))
            cps.append(pltpu.make_async_copy(
                pos4.at[di_v[b].at[j]], dr_v[b].at[pl.ds(j * SB, SB)],
                smg[b]))
        return cps

    def start_scatter(b):
        for j in range(NSB):
            pltpu.async_copy(en_v[b].at[pl.ds(j * SB, SB)],
                             acc_sh.at[si_v[b].at[j]], sms[b], add=True)

    def wait_scatter(b):
        for j in range(NSB):
            pltpu.make_async_copy(en_v[b].at[pl.ds(j * SB, SB)],
                                  acc_sh.at[si_v[b].at[j]], sms[b]).wait()

    def prefetch(ci, b):
        icps = idx_copies(ci, b)
        for cp in icps:
            cp.start()
        for cp in icps:
            cp.wait()
        for cp in gather_copies(b):
            cp.start()

    def compute(b):
        def _group(g, gcarry):
            rid = g * L + iota
            xs = plsc.load_gather(sr_v[b], [rid, col0])
            ys = plsc.load_gather(sr_v[b], [rid, col1])
            zs = plsc.load_gather(sr_v[b], [rid, col2])
            xd = plsc.load_gather(dr_v[b], [rid, col0])
            yd = plsc.load_gather(dr_v[b], [rid, col1])
            zd = plsc.load_gather(dr_v[b], [rid, col2])
            dx = xd - xs
            dy = yd - ys
            dz = zd - zs
            r2 = dx * dx + dy * dy + dz * dz
            t = sig2 / r2
            t3 = t * t * t
            eng = two_eps * (t3 * t3 - t3)
            en_v[b][pl.ds(g * L, L)] = eng
            return gcarry

        lax.fori_loop(0, CHUNK // L, _group, 0)

    prefetch(0, 0)

    def _step(st, carry):
        for b in (0, 1):
            ci = st * 2 + b

            @pl.when(ci >= 1)
            def _():
                wait_scatter(1 - b)

            @pl.when(ci + 1 < N_CHUNKS)
            def _():
                prefetch(ci + 1, 1 - b)

            for cp in gather_copies(b):
                cp.wait()
            compute(b)
            start_scatter(b)
        return carry

    lax.fori_loop(0, N_CHUNKS // 2, _step, 0)
    wait_scatter((N_CHUNKS - 1) % 2)

    plsc.subcore_barrier()
    pltpu.sync_copy(acc_sh.at[pl.ds(s * SLICE, SLICE)], stage_v)
    pltpu.sync_copy(stage_v, out.at[pl.ds(c * NPAD + s * SLICE, SLICE)])


def kernel(pos, edge_index):
    pos4 = jnp.zeros((NPAD, 4), jnp.float32)
    pos4 = pos4.at[:N_NODES, :3].set(pos.astype(jnp.float32))
    pos4 = pos4.at[N_NODES + 1, 0].set(1.0)
    npad_e = EPAD - N_EDGES
    si = jnp.concatenate([
        edge_index[0].astype(jnp.int32),
        jnp.full((npad_e,), N_NODES, jnp.int32),
    ])
    di = jnp.concatenate([
        edge_index[1].astype(jnp.int32),
        jnp.full((npad_e,), N_NODES + 1, jnp.int32),
    ])
    partial = _lj_sc(pos4, si, di)
    return (partial[:N_NODES] + partial[NPAD:NPAD + N_NODES]).reshape(
        N_NODES, 1)


# speedup vs baseline: 14.5075x; 1.0365x over previous
import functools

import jax
import jax.numpy as jnp
from jax import lax
from jax.experimental import pallas as pl
from jax.experimental.pallas import tpu as pltpu
from jax.experimental.pallas import tpu_sc as plsc

LJ_SIGMA = 0.01
LJ_EPSILON = 1.0
N_NODES = 50000
N_EDGES = 1600000

NC, NS, L = 2, 16, 16
NW = NC * NS
NPAD = 50176
SLICE = NPAD // NS
SB = 3200
NSB = 1
CHUNK = SB * NSB
N_CHUNKS = 16
E_PER_W = CHUNK * N_CHUNKS
EPAD = NW * E_PER_W

_mesh = plsc.VectorSubcoreMesh(core_axis_name="c", subcore_axis_name="s")


@functools.partial(
    pl.kernel,
    out_type=jax.ShapeDtypeStruct((NC * NPAD,), jnp.float32),
    mesh=_mesh,
    compiler_params=pltpu.CompilerParams(
        needs_layout_passes=False, use_tc_tiling_on_sc=False),
    scratch_types=[
        pltpu.VMEM((NSB, SB), jnp.int32),
        pltpu.VMEM((NSB, SB), jnp.int32),
        pltpu.VMEM((NSB, SB), jnp.int32),
        pltpu.VMEM((NSB, SB), jnp.int32),
        pltpu.VMEM((CHUNK, 3), jnp.float32),
        pltpu.VMEM((CHUNK, 3), jnp.float32),
        pltpu.VMEM((CHUNK, 3), jnp.float32),
        pltpu.VMEM((CHUNK, 3), jnp.float32),
        pltpu.VMEM((CHUNK,), jnp.float32),
        pltpu.VMEM((CHUNK,), jnp.float32),
        pltpu.VMEM((SLICE,), jnp.float32),
        pltpu.VMEM_SHARED((NPAD,), jnp.float32),
        pltpu.SemaphoreType.DMA,
        pltpu.SemaphoreType.DMA,
        pltpu.SemaphoreType.DMA,
        pltpu.SemaphoreType.DMA,
        pltpu.SemaphoreType.DMA,
        pltpu.SemaphoreType.DMA,
    ],
)
def _lj_sc(pos3, srci, dsti, out, si0, si1, di0, di1, sr0, sr1, dr0, dr1,
           en0, en1, stage_v, acc_sh, smi0, smi1, smg0, smg1, sms0, sms1):
    c = lax.axis_index("c")
    s = lax.axis_index("s")
    wid = c * NS + s

    si_v = (si0, si1)
    di_v = (di0, di1)
    sr_v = (sr0, sr1)
    dr_v = (dr0, dr1)
    en_v = (en0, en1)
    smi = (smi0, smi1)
    smg = (smg0, smg1)
    sms = (sms0, sms1)

    zero16 = jnp.zeros((L,), jnp.float32)

    def _zero(i, carry):
        stage_v[pl.ds(i * L, L)] = zero16
        return carry

    lax.fori_loop(0, SLICE // L, _zero, 0)
    pltpu.sync_copy(stage_v, acc_sh.at[pl.ds(s * SLICE, SLICE)])
    plsc.subcore_barrier()

    iota = lax.iota(jnp.int32, L)
    col0 = jnp.zeros((L,), jnp.int32)
    col1 = jnp.full((L,), 1, jnp.int32)
    col2 = jnp.full((L,), 2, jnp.int32)
    sig2 = jnp.full((L,), LJ_SIGMA * LJ_SIGMA, jnp.float32)
    two_eps = jnp.full((L,), 2.0 * LJ_EPSILON, jnp.float32)

    base_e = wid * E_PER_W

    def idx_copies(ci, b):
        off = base_e + ci * CHUNK
        cps = []
        for j in range(NSB):
            cps.append(pltpu.make_async_copy(
                srci.at[pl.ds(off + j * SB, SB)], si_v[b].at[j], smi[b]))
            cps.append(pltpu.make_async_copy(
                dsti.at[pl.ds(off + j * SB, SB)], di_v[b].at[j], smi[b]))
        return cps

    def gather_copies(b):
        cps = []
        for j in range(NSB):
            cps.append(pltpu.make_async_copy(
                pos3.at[si_v[b].at[j]], sr_v[b].at[pl.ds(j * SB, SB)],
                smg[b]))
            cps.append(pltpu.make_async_copy(
                pos3.at[di_v[b].at[j]], dr_v[b].at[pl.ds(j * SB, SB)],
                smg[b]))
        return cps

    def start_scatter(b):
        for j in range(NSB):
            pltpu.async_copy(en_v[b].at[pl.ds(j * SB, SB)],
                             acc_sh.at[si_v[b].at[j]], sms[b], add=True)

    def wait_scatter(b):
        for j in range(NSB):
            pltpu.make_async_copy(en_v[b].at[pl.ds(j * SB, SB)],
                                  acc_sh.at[si_v[b].at[j]], sms[b]).wait()

    def prefetch(ci, b):
        icps = idx_copies(ci, b)
        for cp in icps:
            cp.start()
        for cp in icps:
            cp.wait()
        for cp in gather_copies(b):
            cp.start()

    def compute(b):
        def _group(g, gcarry):
            rid = g * L + iota
            xs = plsc.load_gather(sr_v[b], [rid, col0])
            ys = plsc.load_gather(sr_v[b], [rid, col1])
            zs = plsc.load_gather(sr_v[b], [rid, col2])
            xd = plsc.load_gather(dr_v[b], [rid, col0])
            yd = plsc.load_gather(dr_v[b], [rid, col1])
            zd = plsc.load_gather(dr_v[b], [rid, col2])
            dx = xd - xs
            dy = yd - ys
            dz = zd - zs
            r2 = dx * dx + dy * dy + dz * dz
            t = sig2 / r2
            t3 = t * t * t
            eng = two_eps * (t3 * t3 - t3)
            en_v[b][pl.ds(g * L, L)] = eng
            return gcarry

        lax.fori_loop(0, CHUNK // L, _group, 0)

    prefetch(0, 0)

    def _step(st, carry):
        for b in (0, 1):
            ci = st * 2 + b

            @pl.when(ci >= 1)
            def _():
                wait_scatter(1 - b)

            @pl.when(ci + 1 < N_CHUNKS)
            def _():
                prefetch(ci + 1, 1 - b)

            for cp in gather_copies(b):
                cp.wait()
            compute(b)
            start_scatter(b)
        return carry

    lax.fori_loop(0, N_CHUNKS // 2, _step, 0)
    wait_scatter((N_CHUNKS - 1) % 2)

    plsc.subcore_barrier()
    pltpu.sync_copy(acc_sh.at[pl.ds(s * SLICE, SLICE)], stage_v)
    pltpu.sync_copy(stage_v, out.at[pl.ds(c * NPAD + s * SLICE, SLICE)])


def kernel(pos, edge_index):
    pad_rows = jnp.zeros((8, 3), jnp.float32).at[1, 0].set(1.0)
    pos3 = jnp.concatenate([pos.astype(jnp.float32), pad_rows])
    npad_e = EPAD - N_EDGES
    si = jnp.concatenate([
        edge_index[0].astype(jnp.int32),
        jnp.full((npad_e,), N_NODES, jnp.int32),
    ])
    di = jnp.concatenate([
        edge_index[1].astype(jnp.int32),
        jnp.full((npad_e,), N_NODES + 1, jnp.int32),
    ])
    partial = _lj_sc(pos3, si, di)
    return (partial[:N_NODES] + partial[NPAD:NPAD + N_NODES]).reshape(
        N_NODES, 1)


# speedup vs baseline: 32.1559x; 2.2165x over previous
import functools

import jax
import jax.numpy as jnp
from jax import lax
from jax.experimental import pallas as pl
from jax.experimental.pallas import tpu as pltpu
from jax.experimental.pallas import tpu_sc as plsc

LJ_SIGMA = 0.01
LJ_EPSILON = 1.0
N_NODES = 50000
N_EDGES = 1600000

NC, NS, L = 2, 16, 16
NW = NC * NS
NPAD = 50176
SLICE = NPAD // NS
CHUNK = 2000
N_CHUNKS = 25
E_PER_W = CHUNK * N_CHUNKS

_mesh = plsc.VectorSubcoreMesh(core_axis_name="c", subcore_axis_name="s")


@functools.partial(
    pl.kernel,
    out_type=jax.ShapeDtypeStruct((NC * NPAD,), jnp.float32),
    mesh=_mesh,
    compiler_params=pltpu.CompilerParams(
        needs_layout_passes=False, use_tc_tiling_on_sc=False),
    scratch_types=[
        pltpu.VMEM((1, CHUNK), jnp.int32),
        pltpu.VMEM((1, CHUNK), jnp.int32),
        pltpu.VMEM((1, CHUNK), jnp.int32),
        pltpu.VMEM((1, CHUNK), jnp.int32),
        pltpu.VMEM((CHUNK, 3), jnp.float32),
        pltpu.VMEM((CHUNK, 3), jnp.float32),
        pltpu.VMEM((CHUNK, 3), jnp.float32),
        pltpu.VMEM((CHUNK, 3), jnp.float32),
        pltpu.VMEM((CHUNK,), jnp.float32),
        pltpu.VMEM((CHUNK,), jnp.float32),
        pltpu.VMEM((SLICE,), jnp.float32),
        pltpu.VMEM_SHARED((NPAD,), jnp.float32),
        pltpu.SemaphoreType.DMA,
        pltpu.SemaphoreType.DMA,
        pltpu.SemaphoreType.DMA,
        pltpu.SemaphoreType.DMA,
        pltpu.SemaphoreType.DMA,
        pltpu.SemaphoreType.DMA,
    ],
)
def _lj_sc(pos3, srci, dsti, out, si0, si1, di0, di1, sr0, sr1, dr0, dr1,
           en0, en1, stage_v, acc_sh, smi0, smi1, smg0, smg1, sms0, sms1):
    c = lax.axis_index("c")
    s = lax.axis_index("s")
    wid = c * NS + s

    si_v = (si0, si1)
    di_v = (di0, di1)
    sr_v = (sr0, sr1)
    dr_v = (dr0, dr1)
    en_v = (en0, en1)
    smi = (smi0, smi1)
    smg = (smg0, smg1)
    sms = (sms0, sms1)

    zero16 = jnp.zeros((L,), jnp.float32)

    def _zero(i, carry):
        stage_v[pl.ds(i * L, L)] = zero16
        return carry

    lax.fori_loop(0, SLICE // L, _zero, 0)
    pltpu.sync_copy(stage_v, acc_sh.at[pl.ds(s * SLICE, SLICE)])
    plsc.subcore_barrier()

    iota = lax.iota(jnp.int32, L)
    col0 = jnp.zeros((L,), jnp.int32)
    col1 = jnp.full((L,), 1, jnp.int32)
    col2 = jnp.full((L,), 2, jnp.int32)
    sig2 = jnp.full((L,), LJ_SIGMA * LJ_SIGMA, jnp.float32)
    two_eps = jnp.full((L,), 2.0 * LJ_EPSILON, jnp.float32)

    base_e = wid * E_PER_W

    def idx_copies(ci, b):
        off = base_e + ci * CHUNK
        return [
            pltpu.make_async_copy(
                srci.at[pl.ds(off, CHUNK)], si_v[b].at[0], smi[b]),
            pltpu.make_async_copy(
                dsti.at[pl.ds(off, CHUNK)], di_v[b].at[0], smi[b]),
        ]

    def gather_copies(b):
        return [
            pltpu.make_async_copy(pos3.at[si_v[b].at[0]], sr_v[b], smg[b]),
            pltpu.make_async_copy(pos3.at[di_v[b].at[0]], dr_v[b], smg[b]),
        ]

    def start_scatter(b):
        pltpu.async_copy(en_v[b], acc_sh.at[si_v[b].at[0]], sms[b], add=True)

    def wait_scatter(b):
        pltpu.make_async_copy(en_v[b], acc_sh.at[si_v[b].at[0]],
                              sms[b]).wait()

    def prefetch(ci, b):
        icps = idx_copies(ci, b)
        for cp in icps:
            cp.start()
        for cp in icps:
            cp.wait()
        for cp in gather_copies(b):
            cp.start()

    def compute(b):
        def _group(g, gcarry):
            rid = g * L + iota
            xs = plsc.load_gather(sr_v[b], [rid, col0])
            ys = plsc.load_gather(sr_v[b], [rid, col1])
            zs = plsc.load_gather(sr_v[b], [rid, col2])
            xd = plsc.load_gather(dr_v[b], [rid, col0])
            yd = plsc.load_gather(dr_v[b], [rid, col1])
            zd = plsc.load_gather(dr_v[b], [rid, col2])
            dx = xd - xs
            dy = yd - ys
            dz = zd - zs
            r2 = dx * dx + dy * dy + dz * dz
            t = sig2 / r2
            t3 = t * t * t
            eng = two_eps * (t3 * t3 - t3)
            en_v[b][pl.ds(g * L, L)] = eng
            return gcarry

        lax.fori_loop(0, CHUNK // L, _group, 0)

    prefetch(0, 0)

    def _step(st, carry):
        for b in (0, 1):
            ci = st * 2 + b

            @pl.when(ci >= 1)
            def _():
                wait_scatter(1 - b)

            prefetch(ci + 1, 1 - b)
            for cp in gather_copies(b):
                cp.wait()
            compute(b)
            start_scatter(b)
        return carry

    lax.fori_loop(0, (N_CHUNKS - 1) // 2, _step, 0)

    wait_scatter(1)
    for cp in gather_copies(0):
        cp.wait()
    compute(0)
    start_scatter(0)
    wait_scatter(0)

    plsc.subcore_barrier()
    pltpu.sync_copy(acc_sh.at[pl.ds(s * SLICE, SLICE)], stage_v)
    pltpu.sync_copy(stage_v, out.at[pl.ds(c * NPAD + s * SLICE, SLICE)])


def kernel(pos, edge_index):
    si = edge_index[0].astype(jnp.int32)
    di = edge_index[1].astype(jnp.int32)
    partial = _lj_sc(pos.astype(jnp.float32), si, di)
    return (partial[:N_NODES] + partial[NPAD:NPAD + N_NODES]).reshape(
        N_NODES, 1)


# speedup vs baseline: 35.5910x; 1.1068x over previous
import functools

import jax
import jax.numpy as jnp
from jax import lax
from jax.experimental import pallas as pl
from jax.experimental.pallas import tpu as pltpu
from jax.experimental.pallas import tpu_sc as plsc

LJ_SIGMA = 0.01
LJ_EPSILON = 1.0
N_NODES = 50000
N_EDGES = 1600000

NC, NS, L = 2, 16, 16
NW = NC * NS
NPAD = 50176
SLICE = NPAD // NS
CHUNK = 2000
N_CHUNKS = 25
E_PER_W = CHUNK * N_CHUNKS

_mesh = plsc.VectorSubcoreMesh(core_axis_name="c", subcore_axis_name="s")


@functools.partial(
    pl.kernel,
    out_type=jax.ShapeDtypeStruct((NC * NPAD,), jnp.float32),
    mesh=_mesh,
    compiler_params=pltpu.CompilerParams(
        needs_layout_passes=False, use_tc_tiling_on_sc=False),
    scratch_types=[
        pltpu.VMEM((1, CHUNK), jnp.int32),
        pltpu.VMEM((1, CHUNK), jnp.int32),
        pltpu.VMEM((1, CHUNK), jnp.int32),
        pltpu.VMEM((1, CHUNK), jnp.int32),
        pltpu.VMEM((CHUNK, 3), jnp.float32),
        pltpu.VMEM((CHUNK, 3), jnp.float32),
        pltpu.VMEM((CHUNK, 3), jnp.float32),
        pltpu.VMEM((CHUNK, 3), jnp.float32),
        pltpu.VMEM((CHUNK,), jnp.float32),
        pltpu.VMEM((CHUNK,), jnp.float32),
        pltpu.VMEM((SLICE,), jnp.float32),
        pltpu.VMEM_SHARED((NPAD,), jnp.float32),
        pltpu.SemaphoreType.DMA,
        pltpu.SemaphoreType.DMA,
        pltpu.SemaphoreType.DMA,
        pltpu.SemaphoreType.DMA,
        pltpu.SemaphoreType.DMA,
        pltpu.SemaphoreType.DMA,
    ],
)
def _lj_sc(pos3, eidx, out, si0, si1, di0, di1, sr0, sr1, dr0, dr1,
           en0, en1, stage_v, acc_sh, smi0, smi1, smg0, smg1, sms0, sms1):
    c = lax.axis_index("c")
    s = lax.axis_index("s")
    wid = c * NS + s

    si_v = (si0, si1)
    di_v = (di0, di1)
    sr_v = (sr0, sr1)
    dr_v = (dr0, dr1)
    en_v = (en0, en1)
    smi = (smi0, smi1)
    smg = (smg0, smg1)
    sms = (sms0, sms1)

    zero16 = jnp.zeros((L,), jnp.float32)

    def _zero(i, carry):
        stage_v[pl.ds(i * L, L)] = zero16
        return carry

    lax.fori_loop(0, SLICE // L, _zero, 0)
    pltpu.sync_copy(stage_v, acc_sh.at[pl.ds(s * SLICE, SLICE)])
    plsc.subcore_barrier()

    iota = lax.iota(jnp.int32, L)
    col0 = jnp.zeros((L,), jnp.int32)
    col1 = jnp.full((L,), 1, jnp.int32)
    col2 = jnp.full((L,), 2, jnp.int32)
    sig2 = jnp.full((L,), LJ_SIGMA * LJ_SIGMA, jnp.float32)
    two_eps = jnp.full((L,), 2.0 * LJ_EPSILON, jnp.float32)

    base_e = wid * E_PER_W

    def idx_copies(ci, b):
        off = base_e + ci * CHUNK
        return [
            pltpu.make_async_copy(
                eidx.at[0, pl.ds(off, CHUNK)], si_v[b].at[0], smi[b]),
            pltpu.make_async_copy(
                eidx.at[1, pl.ds(off, CHUNK)], di_v[b].at[0], smi[b]),
        ]

    def gather_copies(b):
        return [
            pltpu.make_async_copy(pos3.at[si_v[b].at[0]], sr_v[b], smg[b]),
            pltpu.make_async_copy(pos3.at[di_v[b].at[0]], dr_v[b], smg[b]),
        ]

    def start_scatter(b):
        pltpu.async_copy(en_v[b], acc_sh.at[si_v[b].at[0]], sms[b], add=True)

    def wait_scatter(b):
        pltpu.make_async_copy(en_v[b], acc_sh.at[si_v[b].at[0]],
                              sms[b]).wait()

    def prefetch(ci, b):
        icps = idx_copies(ci, b)
        for cp in icps:
            cp.start()
        for cp in icps:
            cp.wait()
        for cp in gather_copies(b):
            cp.start()

    def compute(b):
        @plsc.parallel_loop(0, CHUNK // L, unroll=4)
        def _group(g):
            rid = g * L + iota
            xs = plsc.load_gather(sr_v[b], [rid, col0])
            ys = plsc.load_gather(sr_v[b], [rid, col1])
            zs = plsc.load_gather(sr_v[b], [rid, col2])
            xd = plsc.load_gather(dr_v[b], [rid, col0])
            yd = plsc.load_gather(dr_v[b], [rid, col1])
            zd = plsc.load_gather(dr_v[b], [rid, col2])
            dx = xd - xs
            dy = yd - ys
            dz = zd - zs
            r2 = dx * dx + dy * dy + dz * dz
            t = sig2 / r2
            t3 = t * t * t
            eng = two_eps * (t3 * t3 - t3)
            en_v[b][pl.ds(g * L, L)] = eng

    prefetch(0, 0)

    def _step(st, carry):
        for b in (0, 1):
            ci = st * 2 + b

            @pl.when(ci >= 1)
            def _():
                wait_scatter(1 - b)

            prefetch(ci + 1, 1 - b)
            for cp in gather_copies(b):
                cp.wait()
            compute(b)
            start_scatter(b)
        return carry

    lax.fori_loop(0, (N_CHUNKS - 1) // 2, _step, 0)

    wait_scatter(1)
    for cp in gather_copies(0):
        cp.wait()
    compute(0)
    start_scatter(0)
    wait_scatter(0)

    plsc.subcore_barrier()
    pltpu.sync_copy(acc_sh.at[pl.ds(s * SLICE, SLICE)], stage_v)
    pltpu.sync_copy(stage_v, out.at[pl.ds(c * NPAD + s * SLICE, SLICE)])


def kernel(pos, edge_index):
    partial = _lj_sc(pos.astype(jnp.float32),
                     edge_index.astype(jnp.int32))
    return (partial[:N_NODES] + partial[NPAD:NPAD + N_NODES]).reshape(
        N_NODES, 1)


# speedup vs baseline: 52.8919x; 1.4861x over previous
import functools

import jax
import jax.numpy as jnp
from jax import lax
from jax.experimental import pallas as pl
from jax.experimental.pallas import tpu as pltpu
from jax.experimental.pallas import tpu_sc as plsc

LJ_SIGMA = 0.01
LJ_EPSILON = 1.0
N_NODES = 50000
N_EDGES = 1600000

NC, NS, L = 2, 16, 16
NW = NC * NS
NPAD = 50176
SLICE = NPAD // NS
CHUNK = 2000
N_CHUNKS = 25
E_PER_W = CHUNK * N_CHUNKS

_mesh = plsc.VectorSubcoreMesh(core_axis_name="c", subcore_axis_name="s")


@functools.partial(
    pl.kernel,
    out_type=jax.ShapeDtypeStruct((NC * NPAD,), jnp.float32),
    mesh=_mesh,
    compiler_params=pltpu.CompilerParams(
        needs_layout_passes=False, use_tc_tiling_on_sc=False),
    scratch_types=[
        pltpu.VMEM((1, CHUNK), jnp.int32),
        pltpu.VMEM((1, CHUNK), jnp.int32),
        pltpu.VMEM((1, CHUNK), jnp.int32),
        pltpu.VMEM((1, CHUNK), jnp.int32),
        pltpu.VMEM((CHUNK, 3), jnp.float32),
        pltpu.VMEM((CHUNK, 3), jnp.float32),
        pltpu.VMEM((CHUNK, 3), jnp.float32),
        pltpu.VMEM((CHUNK, 3), jnp.float32),
        pltpu.VMEM((CHUNK,), jnp.float32),
        pltpu.VMEM((CHUNK,), jnp.float32),
        pltpu.VMEM((SLICE,), jnp.float32),
        pltpu.VMEM_SHARED((NPAD,), jnp.float32),
        pltpu.VMEM_SHARED((N_NODES, 3), jnp.float32),
        pltpu.SemaphoreType.DMA,
        pltpu.SemaphoreType.DMA,
        pltpu.SemaphoreType.DMA,
        pltpu.SemaphoreType.DMA,
        pltpu.SemaphoreType.DMA,
        pltpu.SemaphoreType.DMA,
    ],
)
def _lj_sc(pos3, eidx, out, si0, si1, di0, di1, sr0, sr1, dr0, dr1,
           en0, en1, stage_v, acc_sh, pos_sh, smi0, smi1, smg0, smg1,
           sms0, sms1):
    c = lax.axis_index("c")
    s = lax.axis_index("s")
    wid = c * NS + s

    si_v = (si0, si1)
    di_v = (di0, di1)
    sr_v = (sr0, sr1)
    dr_v = (dr0, dr1)
    en_v = (en0, en1)
    smi = (smi0, smi1)
    smg = (smg0, smg1)
    sms = (sms0, sms1)

    zero16 = jnp.zeros((L,), jnp.float32)

    def _zero(i, carry):
        stage_v[pl.ds(i * L, L)] = zero16
        return carry

    lax.fori_loop(0, SLICE // L, _zero, 0)
    pltpu.sync_copy(stage_v, acc_sh.at[pl.ds(s * SLICE, SLICE)])

    @pl.when(s == 0)
    def _():
        pltpu.sync_copy(pos3, pos_sh)

    plsc.subcore_barrier()

    iota = lax.iota(jnp.int32, L)
    col0 = jnp.zeros((L,), jnp.int32)
    col1 = jnp.full((L,), 1, jnp.int32)
    col2 = jnp.full((L,), 2, jnp.int32)
    sig2 = jnp.full((L,), LJ_SIGMA * LJ_SIGMA, jnp.float32)
    two_eps = jnp.full((L,), 2.0 * LJ_EPSILON, jnp.float32)

    base_e = wid * E_PER_W

    def idx_copies(ci, b):
        off = base_e + ci * CHUNK
        return [
            pltpu.make_async_copy(
                eidx.at[pl.ds(off, CHUNK)], si_v[b].at[0], smi[b]),
            pltpu.make_async_copy(
                eidx.at[pl.ds(N_EDGES + off, CHUNK)], di_v[b].at[0],
                smi[b]),
        ]

    def gather_copies(b):
        return [
            pltpu.make_async_copy(pos_sh.at[si_v[b].at[0]], sr_v[b],
                                  smg[b]),
            pltpu.make_async_copy(pos_sh.at[di_v[b].at[0]], dr_v[b],
                                  smg[b]),
        ]

    def start_scatter(b):
        pltpu.async_copy(en_v[b], acc_sh.at[si_v[b].at[0]], sms[b], add=True)

    def wait_scatter(b):
        pltpu.make_async_copy(en_v[b], acc_sh.at[si_v[b].at[0]],
                              sms[b]).wait()

    def prefetch(ci, b):
        icps = idx_copies(ci, b)
        for cp in icps:
            cp.start()
        for cp in icps:
            cp.wait()
        for cp in gather_copies(b):
            cp.start()

    def compute(b):
        @plsc.parallel_loop(0, CHUNK // L, unroll=4)
        def _group(g):
            rid = g * L + iota
            xs = plsc.load_gather(sr_v[b], [rid, col0])
            ys = plsc.load_gather(sr_v[b], [rid, col1])
            zs = plsc.load_gather(sr_v[b], [rid, col2])
            xd = plsc.load_gather(dr_v[b], [rid, col0])
            yd = plsc.load_gather(dr_v[b], [rid, col1])
            zd = plsc.load_gather(dr_v[b], [rid, col2])
            dx = xd - xs
            dy = yd - ys
            dz = zd - zs
            r2 = dx * dx + dy * dy + dz * dz
            t = sig2 / r2
            t3 = t * t * t
            eng = two_eps * (t3 * t3 - t3)
            en_v[b][pl.ds(g * L, L)] = eng

    prefetch(0, 0)

    def _step(st, carry):
        for b in (0, 1):
            ci = st * 2 + b

            @pl.when(ci >= 1)
            def _():
                wait_scatter(1 - b)

            prefetch(ci + 1, 1 - b)
            for cp in gather_copies(b):
                cp.wait()
            compute(b)
            start_scatter(b)
        return carry

    lax.fori_loop(0, (N_CHUNKS - 1) // 2, _step, 0)

    wait_scatter(1)
    for cp in gather_copies(0):
        cp.wait()
    compute(0)
    start_scatter(0)
    wait_scatter(0)

    plsc.subcore_barrier()
    pltpu.sync_copy(acc_sh.at[pl.ds(s * SLICE, SLICE)], stage_v)
    pltpu.sync_copy(stage_v, out.at[pl.ds(c * NPAD + s * SLICE, SLICE)])


def kernel(pos, edge_index):
    eidx_flat = edge_index.astype(jnp.int32).reshape(-1)
    partial = _lj_sc(pos.astype(jnp.float32), eidx_flat)
    return (partial[:N_NODES] + partial[NPAD:NPAD + N_NODES]).reshape(
        N_NODES, 1)
